# baseline (speedup 1.0000x reference)
import jax
import jax.numpy as jnp
from jax.experimental import pallas as pl
from jax.experimental.pallas import tpu as pltpu

N = 4096
F = 64
BM = 512
NB = N // BM


def _sgconv_kernel(x_ref, adj_ref, out_ref, adjbf, hx):
    t = pl.program_id(0)

    @pl.when(t == 0)
    def _init():
        hx[:, 0:F] = jnp.zeros((N, F), jnp.bfloat16)
        hx[:, F:2 * F] = x_ref[...]

    @pl.when(t < NB)
    def _sweep():
        abf = adj_ref[...].astype(jnp.bfloat16)
        adjbf[pl.ds(t * BM, BM), :] = abf
        r = jnp.dot(abf, hx[...], preferred_element_type=jnp.float32)
        out_ref[pl.ds(t * BM, BM), :] = r[:, 0:F]
        hx[pl.ds(t * BM, BM), 0:F] = r[:, F:2 * F].astype(jnp.bfloat16)

    @pl.when(t > 0)
    def _column():
        c = t - 1
        out_ref[...] = out_ref[...] + jnp.dot(
            adjbf[:, pl.ds(c * BM, BM)], hx[pl.ds(c * BM, BM), 0:F],
            preferred_element_type=jnp.float32)


@jax.jit
def kernel(x, adj):
    return pl.pallas_call(
        _sgconv_kernel,
        grid=(NB + 1,),
        in_specs=[
            pl.BlockSpec((N, F), lambda t: (0, 0)),
            pl.BlockSpec((BM, N), lambda t: (jnp.minimum(t, NB - 1), 0)),
        ],
        out_specs=pl.BlockSpec((N, F), lambda t: (0, 0)),
        out_shape=jax.ShapeDtypeStruct((N, F), jnp.float32),
        scratch_shapes=[
            pltpu.VMEM((N, N), jnp.bfloat16),
            pltpu.VMEM((N, 2 * F), jnp.bfloat16),
        ],
    )(x.astype(jnp.bfloat16), adj)


# speedup vs baseline: 1.0007x; 1.0007x over previous
import jax
import jax.numpy as jnp
from jax.experimental import pallas as pl
from jax.experimental.pallas import tpu as pltpu

N = 4096
F = 64
BM = 512
NB = N // BM


def _sgconv_kernel(x_ref, adj_ref, out_ref, adjbf, hx, h1s):
    t = pl.program_id(0)

    @pl.when(t == 0)
    def _init():
        hx[:, 0:F] = jnp.zeros((N, F), jnp.bfloat16)
        hx[:, F:2 * F] = x_ref[...]

    @pl.when(t < NB)
    def _sweep():
        abf = adj_ref[...].astype(jnp.bfloat16)
        adjbf[pl.ds(t * BM, BM), :] = abf
        r = jnp.dot(abf, hx[...], preferred_element_type=jnp.float32)
        out_ref[pl.ds(t * BM, BM), :] = r[:, 0:F]

        @pl.when(t > 0)
        def _column():
            out_ref[...] = out_ref[...] + jnp.dot(
                adjbf[:, pl.ds((t - 1) * BM, BM)], h1s[...],
                preferred_element_type=jnp.float32)
            hx[pl.ds((t - 1) * BM, BM), 0:F] = h1s[...]

        h1s[...] = r[:, F:2 * F].astype(jnp.bfloat16)

    @pl.when(t == NB)
    def _epilogue():
        out_ref[...] = out_ref[...] + jnp.dot(
            adjbf[:, pl.ds((NB - 1) * BM, BM)], h1s[...],
            preferred_element_type=jnp.float32)


@jax.jit
def kernel(x, adj):
    return pl.pallas_call(
        _sgconv_kernel,
        grid=(NB + 1,),
        in_specs=[
            pl.BlockSpec((N, F), lambda t: (0, 0)),
            pl.BlockSpec((BM, N), lambda t: (jnp.minimum(t, NB - 1), 0)),
        ],
        out_specs=pl.BlockSpec((N, F), lambda t: (0, 0)),
        out_shape=jax.ShapeDtypeStruct((N, F), jnp.float32),
        scratch_shapes=[
            pltpu.VMEM((N, N), jnp.bfloat16),
            pltpu.VMEM((N, 2 * F), jnp.bfloat16),
            pltpu.VMEM((BM, F), jnp.bfloat16),
        ],
    )(x.astype(jnp.bfloat16), adj)


# speedup vs baseline: 1.0284x; 1.0277x over previous
import jax
import jax.numpy as jnp
from jax.experimental import pallas as pl
from jax.experimental.pallas import tpu as pltpu

N = 4096
F = 64
BM = 512
NB = N // BM


def _sgconv_kernel(x_ref, adj_ref, out_ref, adjbf, hx, h1s):
    t = pl.program_id(0)

    @pl.when(t == 0)
    def _init():
        hx[:, 0:F] = jnp.zeros((N, F), jnp.bfloat16)
        hx[:, F:2 * F] = x_ref[...]

    @pl.when(t > 0)
    def _column():
        out_ref[...] = out_ref[...] + jnp.dot(
            adjbf[:, pl.ds((t - 1) * BM, BM)], h1s[...],
            preferred_element_type=jnp.float32)

        @pl.when(t < NB)
        def _publish():
            hx[pl.ds((t - 1) * BM, BM), 0:F] = h1s[...]

    @pl.when(t < NB)
    def _sweep():
        abf = adj_ref[...].astype(jnp.bfloat16)
        adjbf[pl.ds(t * BM, BM), :] = abf
        r = jnp.dot(abf, hx[...], preferred_element_type=jnp.float32)
        out_ref[pl.ds(t * BM, BM), :] = r[:, 0:F]
        h1s[...] = r[:, F:2 * F].astype(jnp.bfloat16)


@jax.jit
def kernel(x, adj):
    return pl.pallas_call(
        _sgconv_kernel,
        grid=(NB + 1,),
        in_specs=[
            pl.BlockSpec((N, F), lambda t: (0, 0)),
            pl.BlockSpec((BM, N), lambda t: (jnp.minimum(t, NB - 1), 0)),
        ],
        out_specs=pl.BlockSpec((N, F), lambda t: (0, 0)),
        out_shape=jax.ShapeDtypeStruct((N, F), jnp.float32),
        scratch_shapes=[
            pltpu.VMEM((N, N), jnp.bfloat16),
            pltpu.VMEM((N, 2 * F), jnp.bfloat16),
            pltpu.VMEM((BM, F), jnp.bfloat16),
        ],
    )(x.astype(jnp.bfloat16), adj)
